# baseline (speedup 1.0000x reference)
import jax
import jax.numpy as jnp
from jax import lax
from jax.experimental import pallas as pl
from jax.experimental.pallas import tpu as pltpu
from jax.experimental.pallas import tpu_sc as plsc

B = 4096
L = 100
LP = 128
E = 64
NC = 2
NS = 16
NW = NC * NS
RPW = B // NW
CH = 4
NCH = RPW // CH


def _sc_body(inp_hbm, emb_hbm, out_hbm, emb_v, inp_v, out_v, sem):
    c = lax.axis_index("c")
    s = lax.axis_index("s")
    wid = s * NC + c
    base = wid * RPW

    pltpu.sync_copy(emb_hbm, emb_v)
    pltpu.sync_copy(inp_hbm.at[pl.ds(base, RPW)], inp_v)

    def chunk_body(ci, carry):
        slot = lax.rem(ci, 4)

        @pl.when(ci >= 4)
        def _():
            pltpu.make_async_copy(
                out_v.at[slot], out_hbm.at[pl.ds(base, CH)], sem
            ).wait()

        def l_body(l, carry2):
            col = inp_v[pl.ds(ci * CH, CH), pl.ds(l, 1)]
            row = emb_v[pl.ds(l * E, E)]
            prod = col * row
            out_v[slot, :, :, pl.ds(l * E, E)] = prod.reshape(CH, 1, E)
            return carry2

        lax.fori_loop(0, L, l_body, 0)
        pltpu.make_async_copy(
            out_v.at[slot], out_hbm.at[pl.ds(base + ci * CH, CH)], sem
        ).start()
        return carry

    lax.fori_loop(0, NCH, chunk_body, 0)
    for _ in range(4):
        pltpu.make_async_copy(
            out_v.at[0], out_hbm.at[pl.ds(base, CH)], sem
        ).wait()


def kernel(input, emb_weight):
    inp_p = jnp.concatenate(
        [input, jnp.zeros((B, LP - L), jnp.float32)], axis=1
    )
    emb_flat = emb_weight.reshape(L * E)
    mesh = plsc.VectorSubcoreMesh(core_axis_name="c", subcore_axis_name="s")
    run = pl.kernel(
        _sc_body,
        mesh=mesh,
        compiler_params=pltpu.CompilerParams(use_tc_tiling_on_sc=False),
        out_type=jax.ShapeDtypeStruct((B, 1, L * E), jnp.float32),
        scratch_types=[
            pltpu.VMEM((L * E,), jnp.float32),
            pltpu.VMEM((RPW, LP), jnp.float32),
            pltpu.VMEM((4, CH, 1, L * E), jnp.float32),
            pltpu.SemaphoreType.DMA,
        ],
    )
    return run(inp_p, emb_flat)


# speedup vs baseline: 1.0833x; 1.0833x over previous
import jax
import jax.numpy as jnp
from jax import lax
from jax.experimental import pallas as pl
from jax.experimental.pallas import tpu as pltpu
from jax.experimental.pallas import tpu_sc as plsc

B = 4096
L = 100
LP = 128
E = 64
NC = 2
NS = 16
NW = NC * NS
RPW = B // NW
CH = 8
NCH = RPW // CH


def _sc_body(inp_hbm, emb_hbm, out_hbm, emb_v, inp_v, out_v, sem):
    c = lax.axis_index("c")
    s = lax.axis_index("s")
    wid = s * NC + c
    base = wid * RPW

    pltpu.sync_copy(emb_hbm, emb_v)
    pltpu.sync_copy(inp_hbm.at[pl.ds(base, RPW)], inp_v)

    def chunk_body(ci, carry):
        slot = lax.rem(ci, 2)

        @pl.when(ci >= 2)
        def _():
            pltpu.make_async_copy(
                out_v.at[slot], out_hbm.at[pl.ds(base, CH)], sem
            ).wait()

        def l_body(l, carry2):
            col = inp_v[pl.ds(ci * CH, CH), pl.ds(l, 1)]
            row = emb_v[pl.ds(l * E, E)]
            prod = col * row
            out_v[slot, :, :, pl.ds(l * E, E)] = prod.reshape(CH, 1, E)
            return carry2

        lax.fori_loop(0, L, l_body, 0)
        pltpu.make_async_copy(
            out_v.at[slot], out_hbm.at[pl.ds(base + ci * CH, CH)], sem
        ).start()
        return carry

    lax.fori_loop(0, NCH, chunk_body, 0)
    pltpu.make_async_copy(out_v.at[0], out_hbm.at[pl.ds(base, CH)], sem).wait()
    pltpu.make_async_copy(out_v.at[1], out_hbm.at[pl.ds(base, CH)], sem).wait()


def kernel(input, emb_weight):
    inp_p = jnp.concatenate(
        [input, jnp.zeros((B, LP - L), jnp.float32)], axis=1
    )
    emb_flat = emb_weight.reshape(L * E)
    mesh = plsc.VectorSubcoreMesh(core_axis_name="c", subcore_axis_name="s")
    run = pl.kernel(
        _sc_body,
        mesh=mesh,
        compiler_params=pltpu.CompilerParams(use_tc_tiling_on_sc=False),
        out_type=jax.ShapeDtypeStruct((B, 1, L * E), jnp.float32),
        scratch_types=[
            pltpu.VMEM((L * E,), jnp.float32),
            pltpu.VMEM((RPW, LP), jnp.float32),
            pltpu.VMEM((2, CH, 1, L * E), jnp.float32),
            pltpu.SemaphoreType.DMA,
        ],
    )
    return run(inp_p, emb_flat)
